# baseline (speedup 1.0000x reference)
import jax
import jax.numpy as jnp
from jax import lax
from jax.experimental import pallas as pl
from jax.experimental.pallas import tpu as pltpu

_BM = 256
_NT = (((1,), (1,)), ((), ()))


def _body(L_ref, x_ref, w_ref, b_ref, o_ref, zt_ref, r_ref):
    i = pl.program_id(0)
    d = x_ref.shape[1]

    @pl.when(i == 0)
    def _():
        w1 = w_ref[:, :d]
        w2 = w_ref[:, d:]
        z = lax.dot_general(
            x_ref[...], w2, _NT, preferred_element_type=jnp.float32
        )
        zt_ref[...] = z.T.astype(jnp.bfloat16)
        r_ref[...] = (
            lax.dot_general(
                x_ref[...], w1, _NT, preferred_element_type=jnp.float32
            )
            + b_ref[...]
        )

    acc = lax.dot_general(
        zt_ref[...],
        L_ref[...].astype(jnp.bfloat16),
        _NT,
        preferred_element_type=jnp.float32,
    )
    o_ref[...] = acc.T + r_ref[pl.ds(i * _BM, _BM), :]


@jax.jit
def kernel(L, x, W, b):
    n, d = x.shape
    out = W.shape[0]
    b2 = b.reshape(1, out)

    return pl.pallas_call(
        _body,
        grid=(n // _BM,),
        in_specs=[
            pl.BlockSpec((_BM, n), lambda i: (i, 0)),
            pl.BlockSpec((n, d), lambda i: (0, 0)),
            pl.BlockSpec((out, 2 * d), lambda i: (0, 0)),
            pl.BlockSpec((1, out), lambda i: (0, 0)),
        ],
        out_specs=pl.BlockSpec((_BM, out), lambda i: (i, 0)),
        out_shape=jax.ShapeDtypeStruct((n, out), jnp.float32),
        scratch_shapes=[
            pltpu.VMEM((out, n), jnp.bfloat16),
            pltpu.VMEM((n, out), jnp.float32),
        ],
    )(L, x, W, b2)


# speedup vs baseline: 1.0125x; 1.0125x over previous
import jax
import jax.numpy as jnp
from jax import lax
from jax.experimental import pallas as pl
from jax.experimental.pallas import tpu as pltpu

_BM = 256
_NT = (((1,), (1,)), ((), ()))


def _body(L_ref, x_ref, w_ref, b_ref, o_ref, zt_ref, rt_ref):
    i = pl.program_id(0)
    d = x_ref.shape[1]

    @pl.when(i == 0)
    def _():
        w1 = w_ref[:, :d]
        w2 = w_ref[:, d:]
        zt_ref[...] = lax.dot_general(
            w2, x_ref[...], _NT, preferred_element_type=jnp.float32
        ).astype(jnp.bfloat16)
        rt_ref[...] = (
            lax.dot_general(
                w1, x_ref[...], _NT, preferred_element_type=jnp.float32
            )
            + b_ref[...]
        )

    acc = lax.dot_general(
        zt_ref[...],
        L_ref[...].astype(jnp.bfloat16),
        _NT,
        preferred_element_type=jnp.float32,
    )
    o_ref[...] = (acc + rt_ref[:, pl.ds(i * _BM, _BM)]).T


@jax.jit
def kernel(L, x, W, b):
    n, d = x.shape
    out = W.shape[0]
    b2 = b.reshape(out, 1)

    return pl.pallas_call(
        _body,
        grid=(n // _BM,),
        in_specs=[
            pl.BlockSpec((_BM, n), lambda i: (i, 0)),
            pl.BlockSpec((n, d), lambda i: (0, 0)),
            pl.BlockSpec((out, 2 * d), lambda i: (0, 0)),
            pl.BlockSpec((out, 1), lambda i: (0, 0)),
        ],
        out_specs=pl.BlockSpec((_BM, out), lambda i: (i, 0)),
        out_shape=jax.ShapeDtypeStruct((n, out), jnp.float32),
        scratch_shapes=[
            pltpu.VMEM((out, n), jnp.bfloat16),
            pltpu.VMEM((out, n), jnp.float32),
        ],
    )(L, x, W, b2)


# speedup vs baseline: 1.0707x; 1.0575x over previous
import jax
import jax.numpy as jnp
from jax import lax
from jax.experimental import pallas as pl
from jax.experimental.pallas import tpu as pltpu

_BM = 512
_NT = (((1,), (1,)), ((), ()))


def _body(L_ref, x_ref, w_ref, b_ref, o_ref, zt_ref, rt_ref):
    i = pl.program_id(0)
    d = x_ref.shape[1]

    @pl.when(i == 0)
    def _():
        w1 = w_ref[:, :d]
        w2 = w_ref[:, d:]
        zt_ref[...] = lax.dot_general(
            w2, x_ref[...], _NT, preferred_element_type=jnp.float32
        ).astype(jnp.bfloat16)
        rt_ref[...] = (
            lax.dot_general(
                w1, x_ref[...], _NT, preferred_element_type=jnp.float32
            )
            + b_ref[...]
        )

    acc = lax.dot_general(
        zt_ref[...],
        L_ref[...].astype(jnp.bfloat16),
        _NT,
        preferred_element_type=jnp.float32,
    )
    o_ref[...] = (acc + rt_ref[:, pl.ds(i * _BM, _BM)]).T


@jax.jit
def kernel(L, x, W, b):
    n, d = x.shape
    out = W.shape[0]
    b2 = b.reshape(out, 1)

    return pl.pallas_call(
        _body,
        grid=(n // _BM,),
        in_specs=[
            pl.BlockSpec((_BM, n), lambda i: (i, 0)),
            pl.BlockSpec((n, d), lambda i: (0, 0)),
            pl.BlockSpec((out, 2 * d), lambda i: (0, 0)),
            pl.BlockSpec((out, 1), lambda i: (0, 0)),
        ],
        out_specs=pl.BlockSpec((_BM, out), lambda i: (i, 0)),
        out_shape=jax.ShapeDtypeStruct((n, out), jnp.float32),
        scratch_shapes=[
            pltpu.VMEM((out, n), jnp.bfloat16),
            pltpu.VMEM((out, n), jnp.float32),
        ],
    )(L, x, W, b2)
